# Initial kernel scaffold; baseline (speedup 1.0000x reference)
import jax
import jax.numpy as jnp
from jax.experimental import pallas as pl


def kernel(x, edge_index, jaw_batch, teeth_batch, params):
    raise NotImplementedError("write your pallas kernel here")


# speedup vs baseline: 3.8493x; 3.8493x over previous
import functools

import jax
import jax.numpy as jnp
from jax import lax
from jax.experimental import pallas as pl
from jax.experimental.pallas import tpu as pltpu
from jax.experimental.pallas import tpu_sc as plsc

N = 10000
E = 320000
NW = 32
RANGE = 313
NPAD = NW * RANGE
CAP = 321536
CHA = 2000
RING = 4096
HALF = 2048
CHB = 128

_MESH = dict(core_axis_name="c", subcore_axis_name="s")


def _wid():
    return lax.axis_index("s") * 2 + lax.axis_index("c")


@functools.cache
def _make_bucketize():
    return functools.partial(
        pl.kernel,
        out_type=[
            jax.ShapeDtypeStruct((NW, CAP), jnp.int32),
            jax.ShapeDtypeStruct((NW, CAP), jnp.int32),
            jax.ShapeDtypeStruct((NW, 16), jnp.int32),
        ],
        compiler_params=pltpu.CompilerParams(needs_layout_passes=False),
        mesh=plsc.VectorSubcoreMesh(**_MESH),
        scratch_types=[
            pltpu.VMEM((CHA,), jnp.int32),
            pltpu.VMEM((CHA,), jnp.int32),
            pltpu.VMEM((RING,), jnp.int32),
            pltpu.VMEM((RING,), jnp.int32),
            pltpu.VMEM((16,), jnp.int32),
            pltpu.SemaphoreType.DMA,
        ],
    )(_bucketize_body)


def _bucketize_body(src_hbm, dst_hbm, bsrc, bdl, cnt,
                    src_ch, dst_ch, ring_s, ring_d, cnt_v, sem):
    w = _wid()
    lo = w * RANGE

    def ini(i, _):
        off = pl.multiple_of(i * 16, 16)
        ring_s[pl.ds(off, 16)] = jnp.zeros((16,), jnp.int32)
        ring_d[pl.ds(off, 16)] = jnp.full((16,), RANGE, jnp.int32)
        return 0
    lax.fori_loop(0, RING // 16, ini, 0)

    def flush(do, f):
        half0 = (f // HALF) % 2 == 0
        foff = pl.multiple_of(f, HALF)

        @pl.when(do & half0)
        def _():
            pltpu.sync_copy(ring_s.at[pl.ds(0, HALF)], bsrc.at[w, pl.ds(foff, HALF)])
            pltpu.sync_copy(ring_d.at[pl.ds(0, HALF)], bdl.at[w, pl.ds(foff, HALF)])

        @pl.when(do & jnp.logical_not(half0))
        def _():
            pltpu.sync_copy(ring_s.at[pl.ds(HALF, HALF)], bsrc.at[w, pl.ds(foff, HALF)])
            pltpu.sync_copy(ring_d.at[pl.ds(HALF, HALF)], bdl.at[w, pl.ds(foff, HALF)])
        return jnp.where(do, f + HALF, f)

    def chunk(g, carry):
        cur_v, f = carry
        goff = pl.multiple_of(g * CHA, 8)
        pltpu.sync_copy(src_hbm.at[pl.ds(goff, CHA)], src_ch)
        pltpu.sync_copy(dst_hbm.at[pl.ds(goff, CHA)], dst_ch)

        def inner(i, cv):
            off = pl.multiple_of(i * 16, 16)
            sv = src_ch[pl.ds(off, 16)]
            dv = dst_ch[pl.ds(off, 16)]
            m = (dv >= lo) & (dv < lo + RANGE)
            mi = m.astype(jnp.int32)
            ps = plsc.cumsum(mi)
            pos = (cv + ps - mi) & (RING - 1)
            plsc.store_scatter(ring_s, [pos], sv, mask=m)
            plsc.store_scatter(ring_d, [pos], dv - lo, mask=m)
            return cv + plsc.all_reduce_population_count(m)

        cur_v = lax.fori_loop(0, CHA // 16, inner, cur_v)
        cur_s = jnp.max(cur_v)
        f = flush(cur_s - f >= HALF, f)
        return cur_v, f

    cur_v, f = lax.fori_loop(
        0, E // CHA, chunk, (jnp.zeros((16,), jnp.int32), jnp.int32(0)))
    flush(jnp.max(cur_v) > f, f)
    cnt_v[...] = cur_v
    pltpu.sync_copy(cnt_v, cnt.at[w])


@functools.cache
def _make_segmax(C):
    accw = (RANGE + 1) * C
    crow = 128

    @functools.partial(
        pl.kernel,
        out_type=jax.ShapeDtypeStruct((NPAD * C,), jnp.float32),
        compiler_params=pltpu.CompilerParams(needs_layout_passes=False),
        mesh=plsc.VectorSubcoreMesh(**_MESH),
        scratch_types=[
            pltpu.VMEM((CHB,), jnp.int32),
            pltpu.VMEM((CHB,), jnp.int32),
            pltpu.VMEM((CHB, crow), jnp.float32),
            pltpu.VMEM((accw,), jnp.float32),
            pltpu.VMEM((16,), jnp.int32),
            pltpu.SemaphoreType.DMA,
        ],
    )
    def seg(q_hbm, bsrc, bdl, cnt, out_hbm, idx_v, dl_v, rows_v, acc, nv, sem):
        w = _wid()
        iota = lax.iota(jnp.int32, 16)
        neg = jnp.full((16,), -jnp.inf, jnp.float32)

        def ini(i, _):
            acc[pl.ds(pl.multiple_of(i * 16, 16), 16)] = neg
            return 0
        lax.fori_loop(0, accw // 16, ini, 0)

        pltpu.sync_copy(cnt.at[w], nv)
        n = jnp.max(nv[...])

        def chunk(g, _):
            goff = pl.multiple_of(g * CHB, 8)
            pltpu.sync_copy(bsrc.at[w, pl.ds(goff, CHB)], idx_v)
            pltpu.sync_copy(bdl.at[w, pl.ds(goff, CHB)], dl_v)
            pltpu.async_copy(q_hbm.at[idx_v], rows_v, sem).wait()

            def grp(j, _):
                off = pl.multiple_of(j * 16, 16)
                dl16 = dl_v[pl.ds(off, 16)]
                for jj in range(16):
                    dls = jnp.max(jnp.where(iota == jj, dl16, 0))
                    base = dls * C
                    e = j * 16 + jj
                    for k in range(C // 16):
                        aoff = pl.multiple_of(base + k * 16, 16)
                        r = plsc.load_gather(
                            rows_v, [jnp.full((16,), e, jnp.int32), k * 16 + iota])
                        acc[pl.ds(aoff, 16)] = jnp.maximum(acc[pl.ds(aoff, 16)], r)
                return 0
            lax.fori_loop(0, CHB // 16, grp, 0)
            return 0

        lax.fori_loop(0, (n + CHB - 1) // CHB, chunk, 0)
        ooff = pl.multiple_of(w * RANGE * C, 8)
        pltpu.sync_copy(acc.at[pl.ds(0, RANGE * C)], out_hbm.at[pl.ds(ooff, RANGE * C)])

    return seg


BR = 1000
GRID = N // BR


def _dot(a, b):
    return jnp.dot(a, b, preferred_element_type=jnp.float32)


def _rows_spec(c):
    return pl.BlockSpec((BR, c), lambda i: (i, 0))


def _full_spec(shape):
    return pl.BlockSpec(shape, lambda i: tuple(0 for _ in shape))


def _stage_call(body, in_arrays, row_in_dims, out_dims):
    n_rows = len(row_in_dims)
    in_specs = [_rows_spec(d) for d in row_in_dims]
    in_specs += [_full_spec(a.shape) for a in in_arrays[n_rows:]]
    return pl.pallas_call(
        body,
        grid=(GRID,),
        in_specs=in_specs,
        out_specs=[_rows_spec(d) for d in out_dims],
        out_shape=[jax.ShapeDtypeStruct((N, d), jnp.float32) for d in out_dims],
    )(*in_arrays)


def _tc1(x, w11, b11, w12, b12, bb1, amb1, bg1, g11, bg11, q1, p1, h1):
    h = jnp.maximum(_dot(x[...], w11[...]) + b11[...], 0.)
    h = jnp.maximum(_dot(h, w12[...]) + b12[...], 0.)
    q1[...] = _dot(h, bb1[...])
    p1[...] = _dot(h, amb1[...]) + bg1[...]
    h1[...] = jnp.maximum(_dot(h, g11[...]) + bg11[...], 0.)


def _tc2(s1, p1, h1, ga, gb, bglm, w21, b21, w22, b22, w23, b23,
         bb2, amb2, bg2, g21, bg21, xm2, q2, p2, h2):
    sap = jnp.maximum(p1[...] + s1[...], 0.)
    h = jnp.maximum(_dot(h1[...], ga[...]) + _dot(sap, gb[...]) + bglm[...], 0.)
    h = jnp.maximum(_dot(h, w21[...]) + b21[...], 0.)
    h = jnp.maximum(_dot(h, w22[...]) + b22[...], 0.)
    xm = jnp.maximum(_dot(h, w23[...]) + b23[...], 0.)
    xm2[...] = xm
    q2[...] = _dot(xm, bb2[...])
    p2[...] = _dot(xm, amb2[...]) + bg2[...]
    h2[...] = jnp.maximum(_dot(xm, g21[...]) + bg21[...], 0.)


def _tc3(s2, p2, bb3, amb3, bg3, sap1, q3, p3):
    sp = jnp.maximum(p2[...] + s2[...], 0.)
    sap1[...] = sp
    q3[...] = _dot(sp, bb3[...])
    p3[...] = _dot(sp, amb3[...]) + bg3[...]


def _tc4(s3, p3, h2, sap1, wa, wb, wc, bglm2, xg2):
    sap2 = jnp.maximum(p3[...] + s3[...], 0.)
    xg2[...] = jnp.maximum(
        _dot(h2[...], wa[...]) + _dot(sap1[...], wb[...])
        + _dot(sap2, wc[...]) + bglm2[...], 0.)


def _tc5(xm2, xg2, m31a, m31b, b31, w32, b32, w33, b33, w34, b34, h4):
    h = jnp.maximum(_dot(xm2[...], m31a[...]) + _dot(xg2[...], m31b[...])
                    + b31[...], 0.)
    h = jnp.maximum(_dot(h, w32[...]) + b32[...], 0.)
    h = jnp.maximum(_dot(h, w33[...]) + b33[...], 0.)
    h4[...] = jnp.maximum(_dot(h, w34[...]) + b34[...], 0.)


def _attn_pool(x, seg, gw, gb, nseg):
    gate = _dot(x, gw) + gb
    ids = lax.broadcasted_iota(jnp.int32, (1, nseg), 1)
    mask = (seg == ids).astype(jnp.float32)
    gmax = jnp.max(jnp.where(mask > 0., gate, -jnp.inf), axis=0, keepdims=True)
    m = jnp.where(jnp.isfinite(gmax), gmax, 0.)
    g = jnp.exp(gate - jnp.sum(mask * m, axis=1, keepdims=True))
    denom = jnp.sum(mask * g, axis=0, keepdims=True)
    gn = g / (jnp.sum(mask * denom, axis=1, keepdims=True) + 1e-16)
    pooled = _dot((mask * gn).T, x)
    return mask, pooled


def _pool_jaw(xg, jb, gw, gb, out):
    x = xg[...]
    mask, pooled = _attn_pool(x, jb[...], gw[...], gb[...], 2)
    out[...] = x + _dot(mask, pooled)


def _pool_teeth(h4, tb, gw, gb, out, tf):
    x = h4[...]
    mask, pooled = _attn_pool(x, tb[...], gw[...], gb[...], 32)
    tf[...] = pooled
    out[...] = x + _dot(mask, pooled)


def _head(h5, tf, tb, hmw, hmb, hgw, hgb, tp1, tbb1, tp2, tbb2, tp3, tbb3,
          abrw, abrb, hm_o, efs_o, abr_o, t_o):
    t = _dot(tf[...], tp1[...]) + tbb1[...]
    t = _dot(t, tp2[...]) + tbb2[...]
    t_o[...] = jax.nn.sigmoid(_dot(t, tp3[...]) + tbb3[...])
    hm = jax.nn.sigmoid(_dot(h5[...], hmw[...]) + hmb[...])
    hm_o[...] = hm
    _, ef = _attn_pool(hm, tb[...], hgw[...], hgb[...], 32)
    efs_o[...] = jax.nn.sigmoid(ef)
    abr_o[...] = jax.nn.sigmoid(_dot(ef, abrw[...]) + abrb[...])


def _ungridded_call(body, out_shapes):
    return pl.pallas_call(
        body,
        out_shape=[jax.ShapeDtypeStruct(s, jnp.float32) for s in out_shapes],
    )


def _wt(params, name):
    return params[name + "_W"].T, params[name + "_b"][None, :]


def _conv_wt(params, name, cin):
    wt = params[name + "_W"].T
    a, b = wt[:cin], wt[cin:]
    return b, a - b, params[name + "_b"][None, :]


def kernel(x, edge_index, jaw_batch, teeth_batch, params):
    src = edge_index[0]
    dst = edge_index[1]
    jb = jaw_batch[:, None]
    tb = teeth_batch[:, None]

    w11, b11 = _wt(params, "mlp1_1")
    w12, b12 = _wt(params, "mlp1_2")
    bb1, amb1, bg1 = _conv_wt(params, "gcn1", 64)
    bb1 = jnp.pad(bb1, ((0, 0), (0, 96)))
    g11, bg11 = _wt(params, "glm1_1")
    g12, bglm = _wt(params, "glm1_2")
    ga, gb_ = g12[:32], g12[32:]
    w21, b21 = _wt(params, "mlp2_1")
    w22, b22 = _wt(params, "mlp2_2")
    w23, b23 = _wt(params, "mlp2_3")
    bb2, amb2, bg2 = _conv_wt(params, "gcn2", 256)
    g21, bg21 = _wt(params, "glm2_1")
    bb3, amb3, bg3 = _conv_wt(params, "gcn3", 128)
    g22, bglm2 = _wt(params, "glm2_2")
    wa, wb, wc = g22[:128], g22[128:256], g22[256:]
    jgw, jgb = _wt(params, "jaw_gate")
    m31, b31 = _wt(params, "mlp3_1")
    m31a, m31b = m31[:256], m31[256:]
    w32, b32 = _wt(params, "mlp3_2")
    w33, b33 = _wt(params, "mlp3_3")
    w34, b34 = _wt(params, "mlp3_4")
    tgw, tgb = _wt(params, "teeth_gate")
    tp1, tbb1 = _wt(params, "tp1")
    tp2, tbb2 = _wt(params, "tp2")
    tp3, tbb3 = _wt(params, "tp3")
    hmw, hmb = _wt(params, "hm")
    hgw, hgb = _wt(params, "hm_gate")
    abrw, abrb = _wt(params, "abr")

    bsrc, bdl, cnt = _make_bucketize()(src, dst)
    _seg32 = _make_segmax(32)
    _seg128 = _make_segmax(128)

    q1, p1, h1 = _stage_call(
        _tc1, [x, w11, b11, w12, b12, bb1, amb1, bg1, g11, bg11],
        [15], [128, 32, 32])
    s1 = _seg32(q1, bsrc, bdl, cnt).reshape(NPAD, 32)[:N]

    xm2, q2, p2, h2 = _stage_call(
        _tc2, [s1, p1, h1, ga, gb_, bglm, w21, b21, w22, b22, w23, b23,
               bb2, amb2, bg2, g21, bg21],
        [32, 32, 32], [256, 128, 128, 128])
    s2 = _seg128(q2, bsrc, bdl, cnt).reshape(NPAD, 128)[:N]

    sap1, q3, p3 = _stage_call(
        _tc3, [s2, p2, bb3, amb3, bg3], [128, 128], [128, 128, 128])
    s3 = _seg128(q3, bsrc, bdl, cnt).reshape(NPAD, 128)[:N]

    xg2_pre, = _stage_call(
        _tc4, [s3, p3, h2, sap1, wa, wb, wc, bglm2],
        [128, 128, 128, 128], [256])

    xg2, = pl.pallas_call(
        _pool_jaw,
        out_shape=[jax.ShapeDtypeStruct((N, 256), jnp.float32)],
    )(xg2_pre, jb, jgw, jgb)

    h4, = _stage_call(
        _tc5, [xm2, xg2, m31a, m31b, b31, w32, b32, w33, b33, w34, b34],
        [256, 256], [128])

    h5, tf = pl.pallas_call(
        _pool_teeth,
        out_shape=[jax.ShapeDtypeStruct((N, 128), jnp.float32),
                   jax.ShapeDtypeStruct((32, 128), jnp.float32)],
    )(h4, tb, tgw, tgb)

    hm, efs, abr, t = pl.pallas_call(
        _head,
        out_shape=[jax.ShapeDtypeStruct((N, 5), jnp.float32),
                   jax.ShapeDtypeStruct((32, 5), jnp.float32),
                   jax.ShapeDtypeStruct((32, 1), jnp.float32),
                   jax.ShapeDtypeStruct((32, 3), jnp.float32)],
    )(h5, tf, tb, hmw, hmb, hgw, hgb, tp1, tbb1, tp2, tbb2, tp3, tbb3,
      abrw, abrb)

    landmark_heatmap = hm.T[None]
    landmark_exist_prob = efs.T[None]
    teeth_abrasion_prob = abr.T[None]
    teeth_twisted = t[:, 0].reshape(1, 1, -1)
    teeth_tilted = t[:, 1].reshape(1, 1, -1)
    teeth_ectopic = t[:, 2].reshape(1, 1, -1)
    return (landmark_heatmap, landmark_exist_prob, teeth_abrasion_prob,
            teeth_twisted, teeth_tilted, teeth_ectopic)


# speedup vs baseline: 3.9734x; 1.0322x over previous
import functools

import jax
import jax.numpy as jnp
from jax import lax
from jax.experimental import pallas as pl
from jax.experimental.pallas import tpu as pltpu
from jax.experimental.pallas import tpu_sc as plsc

N = 10000
E = 320000
NW = 32
RANGE = 313
NPAD = NW * RANGE
CAP = 321536
CHA = 2000
RING = 4096
HALF = 2048
CHB = 128

_MESH = dict(core_axis_name="c", subcore_axis_name="s")


def _wid():
    return lax.axis_index("s") * 2 + lax.axis_index("c")


@functools.cache
def _make_bucketize():
    return functools.partial(
        pl.kernel,
        out_type=[
            jax.ShapeDtypeStruct((NW, CAP), jnp.int32),
            jax.ShapeDtypeStruct((NW, CAP), jnp.int32),
            jax.ShapeDtypeStruct((NW, 16), jnp.int32),
        ],
        compiler_params=pltpu.CompilerParams(needs_layout_passes=False),
        mesh=plsc.VectorSubcoreMesh(**_MESH),
        scratch_types=[
            pltpu.VMEM((CHA,), jnp.int32),
            pltpu.VMEM((CHA,), jnp.int32),
            pltpu.VMEM((RING,), jnp.int32),
            pltpu.VMEM((RING,), jnp.int32),
            pltpu.VMEM((16,), jnp.int32),
            pltpu.VMEM((16,), jnp.int32),
            pltpu.VMEM((16,), jnp.int32),
            pltpu.SemaphoreType.DMA,
        ],
    )(_bucketize_body)


def _bucketize_body(src_hbm, dst_hbm, bsrc, bdl, cnt,
                    src_ch, dst_ch, ring_s, ring_d, cnt_v, stage_s, stage_d, sem):
    w = _wid()
    lo = w * RANGE

    def ini(i, _):
        off = pl.multiple_of(i * 16, 16)
        ring_s[pl.ds(off, 16)] = jnp.zeros((16,), jnp.int32)
        ring_d[pl.ds(off, 16)] = jnp.full((16,), RANGE, jnp.int32)
        return 0
    lax.fori_loop(0, RING // 16, ini, 0)

    def flush(do, f):
        half0 = (f // HALF) % 2 == 0
        foff = pl.multiple_of(f, HALF)

        @pl.when(do & half0)
        def _():
            pltpu.sync_copy(ring_s.at[pl.ds(0, HALF)], bsrc.at[w, pl.ds(foff, HALF)])
            pltpu.sync_copy(ring_d.at[pl.ds(0, HALF)], bdl.at[w, pl.ds(foff, HALF)])

        @pl.when(do & jnp.logical_not(half0))
        def _():
            pltpu.sync_copy(ring_s.at[pl.ds(HALF, HALF)], bsrc.at[w, pl.ds(foff, HALF)])
            pltpu.sync_copy(ring_d.at[pl.ds(HALF, HALF)], bdl.at[w, pl.ds(foff, HALF)])
        return jnp.where(do, f + HALF, f)

    def chunk(g, carry):
        cur_v, f = carry
        goff = pl.multiple_of(g * CHA, 8)
        pltpu.sync_copy(src_hbm.at[pl.ds(goff, CHA)], src_ch)
        pltpu.sync_copy(dst_hbm.at[pl.ds(goff, CHA)], dst_ch)

        lane = lax.iota(jnp.int32, 16)

        def inner(i, cv):
            off = pl.multiple_of(i * 16, 16)
            sv = src_ch[pl.ds(off, 16)]
            dv = dst_ch[pl.ds(off, 16)]
            m = (dv >= lo) & (dv < lo + RANGE)
            cnt16 = plsc.all_reduce_population_count(m)

            @pl.when(cnt16[0] > 0)
            def _():
                plsc.store_compressed(stage_s.at[...], sv, mask=m)
                plsc.store_compressed(stage_d.at[...], dv - lo, mask=m)
                mm = lane < cnt16
                pos = (cv + lane) & (RING - 1)
                plsc.store_scatter(ring_s, [pos], stage_s[...], mask=mm)
                plsc.store_scatter(ring_d, [pos], stage_d[...], mask=mm)
            return cv + cnt16

        cur_v = lax.fori_loop(0, CHA // 16, inner, cur_v)
        cur_s = jnp.max(cur_v)
        f = flush(cur_s - f >= HALF, f)
        return cur_v, f

    cur_v, f = lax.fori_loop(
        0, E // CHA, chunk, (jnp.zeros((16,), jnp.int32), jnp.int32(0)))
    flush(jnp.max(cur_v) > f, f)
    cnt_v[...] = cur_v
    pltpu.sync_copy(cnt_v, cnt.at[w])


@functools.cache
def _make_segmax(C):
    accw = (RANGE + 1) * C
    crow = 128

    @functools.partial(
        pl.kernel,
        out_type=jax.ShapeDtypeStruct((NPAD * C,), jnp.float32),
        compiler_params=pltpu.CompilerParams(needs_layout_passes=False),
        mesh=plsc.VectorSubcoreMesh(**_MESH),
        scratch_types=[
            pltpu.VMEM((CHB,), jnp.int32),
            pltpu.VMEM((CHB,), jnp.int32),
            pltpu.VMEM((CHB,), jnp.int32),
            pltpu.VMEM((CHB, crow), jnp.float32),
            pltpu.VMEM((CHB, crow), jnp.float32),
            pltpu.VMEM((accw,), jnp.float32),
            pltpu.VMEM((16,), jnp.int32),
            pltpu.SemaphoreType.DMA,
            pltpu.SemaphoreType.DMA,
        ],
    )
    def seg(q_hbm, bsrc, bdl, cnt, out_hbm,
            idx0, idx1, dl_v, rows0, rows1, acc, nv, sem0, sem1):
        w = _wid()
        neg = jnp.full((16,), -jnp.inf, jnp.float32)

        def ini(i, _):
            acc[pl.ds(pl.multiple_of(i * 16, 16), 16)] = neg
            return 0
        lax.fori_loop(0, accw // 16, ini, 0)

        pltpu.sync_copy(cnt.at[w], nv)
        n = jnp.max(nv[...])
        nch = (n + CHB - 1) // CHB

        def fetch(h, idx_b, rows_b, sem_b):
            @pl.when(h < nch)
            def _():
                hoff = pl.multiple_of(h * CHB, 8)
                pltpu.sync_copy(bsrc.at[w, pl.ds(hoff, CHB)], idx_b)
                pltpu.async_copy(q_hbm.at[idx_b], rows_b, sem_b)

        def process(g, idx_b, rows_b, sem_b):
            goff = pl.multiple_of(g * CHB, 8)
            pltpu.sync_copy(bdl.at[w, pl.ds(goff, CHB)], dl_v)
            pltpu.make_async_copy(q_hbm.at[idx_b], rows_b, sem_b).wait()

            def grp(j, _):
                off = pl.multiple_of(j * 16, 16)
                dl16 = dl_v[pl.ds(off, 16)]
                for jj in range(16):
                    base = pl.multiple_of(dl16[jj] * C, C)
                    e = j * 16 + jj
                    for k in range(C // 16):
                        aoff = pl.multiple_of(base + k * 16, 16)
                        r = rows_b[e, pl.ds(k * 16, 16)]
                        acc[pl.ds(aoff, 16)] = jnp.maximum(acc[pl.ds(aoff, 16)], r)
                return 0
            lax.fori_loop(0, CHB // 16, grp, 0)

        fetch(jnp.int32(0), idx0, rows0, sem0)

        def pair(g2, _):
            g = g2 * 2

            @pl.when(g < nch)
            def _():
                fetch(g + 1, idx1, rows1, sem1)
                process(g, idx0, rows0, sem0)

            @pl.when(g + 1 < nch)
            def _():
                fetch(g + 2, idx0, rows0, sem0)
                process(g + 1, idx1, rows1, sem1)
            return 0

        lax.fori_loop(0, (nch + 1) // 2, pair, 0)
        ooff = pl.multiple_of(w * RANGE * C, 8)
        pltpu.sync_copy(acc.at[pl.ds(0, RANGE * C)], out_hbm.at[pl.ds(ooff, RANGE * C)])

    return seg


BR = 1000
GRID = N // BR


def _dot(a, b):
    return jnp.dot(a, b, preferred_element_type=jnp.float32)


def _rows_spec(c):
    return pl.BlockSpec((BR, c), lambda i: (i, 0))


def _full_spec(shape):
    return pl.BlockSpec(shape, lambda i: tuple(0 for _ in shape))


def _stage_call(body, in_arrays, row_in_dims, out_dims):
    n_rows = len(row_in_dims)
    in_specs = [_rows_spec(d) for d in row_in_dims]
    in_specs += [_full_spec(a.shape) for a in in_arrays[n_rows:]]
    return pl.pallas_call(
        body,
        grid=(GRID,),
        in_specs=in_specs,
        out_specs=[_rows_spec(d) for d in out_dims],
        out_shape=[jax.ShapeDtypeStruct((N, d), jnp.float32) for d in out_dims],
    )(*in_arrays)


def _tc1(x, w11, b11, w12, b12, bb1, amb1, bg1, g11, bg11, q1, p1, h1):
    h = jnp.maximum(_dot(x[...], w11[...]) + b11[...], 0.)
    h = jnp.maximum(_dot(h, w12[...]) + b12[...], 0.)
    q1[...] = _dot(h, bb1[...])
    p1[...] = _dot(h, amb1[...]) + bg1[...]
    h1[...] = jnp.maximum(_dot(h, g11[...]) + bg11[...], 0.)


def _tc2(s1, p1, h1, ga, gb, bglm, w21, b21, w22, b22, w23, b23,
         bb2, amb2, bg2, g21, bg21, xm2, q2, p2, h2):
    sap = jnp.maximum(p1[...] + s1[...], 0.)
    h = jnp.maximum(_dot(h1[...], ga[...]) + _dot(sap, gb[...]) + bglm[...], 0.)
    h = jnp.maximum(_dot(h, w21[...]) + b21[...], 0.)
    h = jnp.maximum(_dot(h, w22[...]) + b22[...], 0.)
    xm = jnp.maximum(_dot(h, w23[...]) + b23[...], 0.)
    xm2[...] = xm
    q2[...] = _dot(xm, bb2[...])
    p2[...] = _dot(xm, amb2[...]) + bg2[...]
    h2[...] = jnp.maximum(_dot(xm, g21[...]) + bg21[...], 0.)


def _tc3(s2, p2, bb3, amb3, bg3, sap1, q3, p3):
    sp = jnp.maximum(p2[...] + s2[...], 0.)
    sap1[...] = sp
    q3[...] = _dot(sp, bb3[...])
    p3[...] = _dot(sp, amb3[...]) + bg3[...]


def _tc4(s3, p3, h2, sap1, wa, wb, wc, bglm2, xg2):
    sap2 = jnp.maximum(p3[...] + s3[...], 0.)
    xg2[...] = jnp.maximum(
        _dot(h2[...], wa[...]) + _dot(sap1[...], wb[...])
        + _dot(sap2, wc[...]) + bglm2[...], 0.)


def _tc5(xm2, xg2, m31a, m31b, b31, w32, b32, w33, b33, w34, b34, h4):
    h = jnp.maximum(_dot(xm2[...], m31a[...]) + _dot(xg2[...], m31b[...])
                    + b31[...], 0.)
    h = jnp.maximum(_dot(h, w32[...]) + b32[...], 0.)
    h = jnp.maximum(_dot(h, w33[...]) + b33[...], 0.)
    h4[...] = jnp.maximum(_dot(h, w34[...]) + b34[...], 0.)


def _attn_pool(x, seg, gw, gb, nseg):
    gate = _dot(x, gw) + gb
    ids = lax.broadcasted_iota(jnp.int32, (1, nseg), 1)
    mask = (seg == ids).astype(jnp.float32)
    gmax = jnp.max(jnp.where(mask > 0., gate, -jnp.inf), axis=0, keepdims=True)
    m = jnp.where(jnp.isfinite(gmax), gmax, 0.)
    g = jnp.exp(gate - jnp.sum(mask * m, axis=1, keepdims=True))
    denom = jnp.sum(mask * g, axis=0, keepdims=True)
    gn = g / (jnp.sum(mask * denom, axis=1, keepdims=True) + 1e-16)
    pooled = _dot((mask * gn).T, x)
    return mask, pooled


def _pool_jaw(xg, jb, gw, gb, out):
    x = xg[...]
    mask, pooled = _attn_pool(x, jb[...], gw[...], gb[...], 2)
    out[...] = x + _dot(mask, pooled)


def _pool_teeth(h4, tb, gw, gb, out, tf):
    x = h4[...]
    mask, pooled = _attn_pool(x, tb[...], gw[...], gb[...], 32)
    tf[...] = pooled
    out[...] = x + _dot(mask, pooled)


def _head(h5, tf, tb, hmw, hmb, hgw, hgb, tp1, tbb1, tp2, tbb2, tp3, tbb3,
          abrw, abrb, hm_o, efs_o, abr_o, t_o):
    t = _dot(tf[...], tp1[...]) + tbb1[...]
    t = _dot(t, tp2[...]) + tbb2[...]
    t_o[...] = jax.nn.sigmoid(_dot(t, tp3[...]) + tbb3[...])
    hm = jax.nn.sigmoid(_dot(h5[...], hmw[...]) + hmb[...])
    hm_o[...] = hm
    _, ef = _attn_pool(hm, tb[...], hgw[...], hgb[...], 32)
    efs_o[...] = jax.nn.sigmoid(ef)
    abr_o[...] = jax.nn.sigmoid(_dot(ef, abrw[...]) + abrb[...])


def _ungridded_call(body, out_shapes):
    return pl.pallas_call(
        body,
        out_shape=[jax.ShapeDtypeStruct(s, jnp.float32) for s in out_shapes],
    )


def _wt(params, name):
    return params[name + "_W"].T, params[name + "_b"][None, :]


def _conv_wt(params, name, cin):
    wt = params[name + "_W"].T
    a, b = wt[:cin], wt[cin:]
    return b, a - b, params[name + "_b"][None, :]


def kernel(x, edge_index, jaw_batch, teeth_batch, params):
    src = edge_index[0]
    dst = edge_index[1]
    jb = jaw_batch[:, None]
    tb = teeth_batch[:, None]

    w11, b11 = _wt(params, "mlp1_1")
    w12, b12 = _wt(params, "mlp1_2")
    bb1, amb1, bg1 = _conv_wt(params, "gcn1", 64)
    bb1 = jnp.pad(bb1, ((0, 0), (0, 96)))
    g11, bg11 = _wt(params, "glm1_1")
    g12, bglm = _wt(params, "glm1_2")
    ga, gb_ = g12[:32], g12[32:]
    w21, b21 = _wt(params, "mlp2_1")
    w22, b22 = _wt(params, "mlp2_2")
    w23, b23 = _wt(params, "mlp2_3")
    bb2, amb2, bg2 = _conv_wt(params, "gcn2", 256)
    g21, bg21 = _wt(params, "glm2_1")
    bb3, amb3, bg3 = _conv_wt(params, "gcn3", 128)
    g22, bglm2 = _wt(params, "glm2_2")
    wa, wb, wc = g22[:128], g22[128:256], g22[256:]
    jgw, jgb = _wt(params, "jaw_gate")
    m31, b31 = _wt(params, "mlp3_1")
    m31a, m31b = m31[:256], m31[256:]
    w32, b32 = _wt(params, "mlp3_2")
    w33, b33 = _wt(params, "mlp3_3")
    w34, b34 = _wt(params, "mlp3_4")
    tgw, tgb = _wt(params, "teeth_gate")
    tp1, tbb1 = _wt(params, "tp1")
    tp2, tbb2 = _wt(params, "tp2")
    tp3, tbb3 = _wt(params, "tp3")
    hmw, hmb = _wt(params, "hm")
    hgw, hgb = _wt(params, "hm_gate")
    abrw, abrb = _wt(params, "abr")

    bsrc, bdl, cnt = _make_bucketize()(src, dst)
    _seg32 = _make_segmax(32)
    _seg128 = _make_segmax(128)

    q1, p1, h1 = _stage_call(
        _tc1, [x, w11, b11, w12, b12, bb1, amb1, bg1, g11, bg11],
        [15], [128, 32, 32])
    s1 = _seg32(q1, bsrc, bdl, cnt).reshape(NPAD, 32)[:N]

    xm2, q2, p2, h2 = _stage_call(
        _tc2, [s1, p1, h1, ga, gb_, bglm, w21, b21, w22, b22, w23, b23,
               bb2, amb2, bg2, g21, bg21],
        [32, 32, 32], [256, 128, 128, 128])
    s2 = _seg128(q2, bsrc, bdl, cnt).reshape(NPAD, 128)[:N]

    sap1, q3, p3 = _stage_call(
        _tc3, [s2, p2, bb3, amb3, bg3], [128, 128], [128, 128, 128])
    s3 = _seg128(q3, bsrc, bdl, cnt).reshape(NPAD, 128)[:N]

    xg2_pre, = _stage_call(
        _tc4, [s3, p3, h2, sap1, wa, wb, wc, bglm2],
        [128, 128, 128, 128], [256])

    xg2, = pl.pallas_call(
        _pool_jaw,
        out_shape=[jax.ShapeDtypeStruct((N, 256), jnp.float32)],
    )(xg2_pre, jb, jgw, jgb)

    h4, = _stage_call(
        _tc5, [xm2, xg2, m31a, m31b, b31, w32, b32, w33, b33, w34, b34],
        [256, 256], [128])

    h5, tf = pl.pallas_call(
        _pool_teeth,
        out_shape=[jax.ShapeDtypeStruct((N, 128), jnp.float32),
                   jax.ShapeDtypeStruct((32, 128), jnp.float32)],
    )(h4, tb, tgw, tgb)

    hm, efs, abr, t = pl.pallas_call(
        _head,
        out_shape=[jax.ShapeDtypeStruct((N, 5), jnp.float32),
                   jax.ShapeDtypeStruct((32, 5), jnp.float32),
                   jax.ShapeDtypeStruct((32, 1), jnp.float32),
                   jax.ShapeDtypeStruct((32, 3), jnp.float32)],
    )(h5, tf, tb, hmw, hmb, hgw, hgb, tp1, tbb1, tp2, tbb2, tp3, tbb3,
      abrw, abrb)

    landmark_heatmap = hm.T[None]
    landmark_exist_prob = efs.T[None]
    teeth_abrasion_prob = abr.T[None]
    teeth_twisted = t[:, 0].reshape(1, 1, -1)
    teeth_tilted = t[:, 1].reshape(1, 1, -1)
    teeth_ectopic = t[:, 2].reshape(1, 1, -1)
    return (landmark_heatmap, landmark_exist_prob, teeth_abrasion_prob,
            teeth_twisted, teeth_tilted, teeth_ectopic)


# speedup vs baseline: 4.5049x; 1.1338x over previous
import functools

import jax
import jax.numpy as jnp
from jax import lax
from jax.experimental import pallas as pl
from jax.experimental.pallas import tpu as pltpu
from jax.experimental.pallas import tpu_sc as plsc

N = 10000
E = 320000
NW = 32
RANGE = 313
NPAD = NW * RANGE
CAP = 321536
CHA = 1600
RING = 4096
HALF = 2048
CHB = 128

_MESH = dict(core_axis_name="c", subcore_axis_name="s")


def _wid():
    return lax.axis_index("s") * 2 + lax.axis_index("c")


@functools.cache
def _make_bucketize():
    return functools.partial(
        pl.kernel,
        out_type=[
            jax.ShapeDtypeStruct((NW, CAP), jnp.int32),
            jax.ShapeDtypeStruct((NW, CAP), jnp.int32),
            jax.ShapeDtypeStruct((NW, 16), jnp.int32),
        ],
        compiler_params=pltpu.CompilerParams(needs_layout_passes=False),
        mesh=plsc.VectorSubcoreMesh(**_MESH),
        scratch_types=[
            pltpu.VMEM((CHA,), jnp.int32),
            pltpu.VMEM((CHA,), jnp.int32),
            pltpu.VMEM((RING,), jnp.int32),
            pltpu.VMEM((RING,), jnp.int32),
            pltpu.VMEM((16,), jnp.int32),
            pltpu.VMEM((16,), jnp.int32),
            pltpu.VMEM((16,), jnp.int32),
            pltpu.SemaphoreType.DMA,
        ],
    )(_bucketize_body)


def _bucketize_body(src_hbm, dst_hbm, bsrc, bdl, cnt,
                    src_ch, dst_ch, ring_s, ring_d, cnt_v, stage_s, stage_d, sem):
    w = _wid()
    lo = w * RANGE

    def ini(i, _):
        off = pl.multiple_of(i * 16, 16)
        ring_s[pl.ds(off, 16)] = jnp.zeros((16,), jnp.int32)
        ring_d[pl.ds(off, 16)] = jnp.full((16,), RANGE, jnp.int32)
        return 0
    lax.fori_loop(0, RING // 16, ini, 0)

    def flush(do, f):
        half0 = (f // HALF) % 2 == 0
        foff = pl.multiple_of(f, HALF)

        @pl.when(do & half0)
        def _():
            pltpu.sync_copy(ring_s.at[pl.ds(0, HALF)], bsrc.at[w, pl.ds(foff, HALF)])
            pltpu.sync_copy(ring_d.at[pl.ds(0, HALF)], bdl.at[w, pl.ds(foff, HALF)])

        @pl.when(do & jnp.logical_not(half0))
        def _():
            pltpu.sync_copy(ring_s.at[pl.ds(HALF, HALF)], bsrc.at[w, pl.ds(foff, HALF)])
            pltpu.sync_copy(ring_d.at[pl.ds(HALF, HALF)], bdl.at[w, pl.ds(foff, HALF)])
        return jnp.where(do, f + HALF, f)

    def chunk(g, carry):
        cur_v, f = carry
        goff = pl.multiple_of(g * CHA, 8)
        pltpu.sync_copy(src_hbm.at[pl.ds(goff, CHA)], src_ch)
        pltpu.sync_copy(dst_hbm.at[pl.ds(goff, CHA)], dst_ch)

        def inner(i, cv):
            for u in range(4):
                off = pl.multiple_of(i * 64 + u * 16, 16)
                sv = src_ch[pl.ds(off, 16)]
                dv = dst_ch[pl.ds(off, 16)]
                dvl = dv - lo
                m = dvl.astype(jnp.uint32) < jnp.uint32(RANGE)
                mi = m.astype(jnp.int32)
                ps = plsc.cumsum(mi)
                pos = (cv + ps - mi) & (RING - 1)
                plsc.store_scatter(ring_s, [pos], sv, mask=m)
                plsc.store_scatter(ring_d, [pos], dvl, mask=m)
                cv = cv + plsc.all_reduce_population_count(m)
            return cv

        cur_v = lax.fori_loop(0, CHA // 64, inner, cur_v)
        cur_s = jnp.max(cur_v)
        f = flush(cur_s - f >= HALF, f)
        return cur_v, f

    cur_v, f = lax.fori_loop(
        0, E // CHA, chunk, (jnp.zeros((16,), jnp.int32), jnp.int32(0)))
    flush(jnp.max(cur_v) > f, f)
    cnt_v[...] = cur_v
    pltpu.sync_copy(cnt_v, cnt.at[w])


@functools.cache
def _make_segmax(C):
    accw = (RANGE + 1) * C
    crow = 128

    @functools.partial(
        pl.kernel,
        out_type=jax.ShapeDtypeStruct((NPAD * C,), jnp.float32),
        compiler_params=pltpu.CompilerParams(needs_layout_passes=False),
        mesh=plsc.VectorSubcoreMesh(**_MESH),
        scratch_types=[
            pltpu.VMEM((CHB,), jnp.int32),
            pltpu.VMEM((CHB,), jnp.int32),
            pltpu.VMEM((CHB,), jnp.int32),
            pltpu.VMEM((CHB, crow), jnp.float32),
            pltpu.VMEM((CHB, crow), jnp.float32),
            pltpu.VMEM((accw,), jnp.float32),
            pltpu.VMEM((16,), jnp.int32),
            pltpu.SemaphoreType.DMA,
            pltpu.SemaphoreType.DMA,
        ],
    )
    def seg(q_hbm, bsrc, bdl, cnt, out_hbm,
            idx0, idx1, dl_v, rows0, rows1, acc, nv, sem0, sem1):
        w = _wid()
        neg = jnp.full((16,), -jnp.inf, jnp.float32)

        def ini(i, _):
            acc[pl.ds(pl.multiple_of(i * 16, 16), 16)] = neg
            return 0
        lax.fori_loop(0, accw // 16, ini, 0)

        pltpu.sync_copy(cnt.at[w], nv)
        n = jnp.max(nv[...])
        nch = (n + CHB - 1) // CHB

        def fetch(h, idx_b, rows_b, sem_b):
            @pl.when(h < nch)
            def _():
                hoff = pl.multiple_of(h * CHB, 8)
                pltpu.sync_copy(bsrc.at[w, pl.ds(hoff, CHB)], idx_b)
                pltpu.async_copy(q_hbm.at[idx_b], rows_b, sem_b)

        def process(g, idx_b, rows_b, sem_b):
            goff = pl.multiple_of(g * CHB, 8)
            pltpu.sync_copy(bdl.at[w, pl.ds(goff, CHB)], dl_v)
            pltpu.make_async_copy(q_hbm.at[idx_b], rows_b, sem_b).wait()

            def grp(j, _):
                off = pl.multiple_of(j * 16, 16)
                dl16 = dl_v[pl.ds(off, 16)]
                for jj in range(16):
                    base = pl.multiple_of(dl16[jj] * C, C)
                    e = j * 16 + jj
                    for k in range(C // 16):
                        aoff = pl.multiple_of(base + k * 16, 16)
                        r = rows_b[e, pl.ds(k * 16, 16)]
                        acc[pl.ds(aoff, 16)] = jnp.maximum(acc[pl.ds(aoff, 16)], r)
                return 0
            lax.fori_loop(0, CHB // 16, grp, 0)

        fetch(jnp.int32(0), idx0, rows0, sem0)

        def pair(g2, _):
            g = g2 * 2

            @pl.when(g < nch)
            def _():
                fetch(g + 1, idx1, rows1, sem1)
                process(g, idx0, rows0, sem0)

            @pl.when(g + 1 < nch)
            def _():
                fetch(g + 2, idx0, rows0, sem0)
                process(g + 1, idx1, rows1, sem1)
            return 0

        lax.fori_loop(0, (nch + 1) // 2, pair, 0)
        ooff = pl.multiple_of(w * RANGE * C, 8)
        pltpu.sync_copy(acc.at[pl.ds(0, RANGE * C)], out_hbm.at[pl.ds(ooff, RANGE * C)])

    return seg


BR = 1000
GRID = N // BR


def _dot(a, b):
    return jnp.dot(a, b, preferred_element_type=jnp.float32)


def _rows_spec(c):
    return pl.BlockSpec((BR, c), lambda i: (i, 0))


def _full_spec(shape):
    return pl.BlockSpec(shape, lambda i: tuple(0 for _ in shape))


def _stage_call(body, in_arrays, row_in_dims, out_dims):
    n_rows = len(row_in_dims)
    in_specs = [_rows_spec(d) for d in row_in_dims]
    in_specs += [_full_spec(a.shape) for a in in_arrays[n_rows:]]
    return pl.pallas_call(
        body,
        grid=(GRID,),
        in_specs=in_specs,
        out_specs=[_rows_spec(d) for d in out_dims],
        out_shape=[jax.ShapeDtypeStruct((N, d), jnp.float32) for d in out_dims],
    )(*in_arrays)


def _tc1(x, w11, b11, w12, b12, bb1, amb1, bg1, g11, bg11, q1, p1, h1):
    h = jnp.maximum(_dot(x[...], w11[...]) + b11[...], 0.)
    h = jnp.maximum(_dot(h, w12[...]) + b12[...], 0.)
    q1[...] = _dot(h, bb1[...])
    p1[...] = _dot(h, amb1[...]) + bg1[...]
    h1[...] = jnp.maximum(_dot(h, g11[...]) + bg11[...], 0.)


def _tc2(s1, p1, h1, ga, gb, bglm, w21, b21, w22, b22, w23, b23,
         bb2, amb2, bg2, g21, bg21, xm2, q2, p2, h2):
    sap = jnp.maximum(p1[...] + s1[...], 0.)
    h = jnp.maximum(_dot(h1[...], ga[...]) + _dot(sap, gb[...]) + bglm[...], 0.)
    h = jnp.maximum(_dot(h, w21[...]) + b21[...], 0.)
    h = jnp.maximum(_dot(h, w22[...]) + b22[...], 0.)
    xm = jnp.maximum(_dot(h, w23[...]) + b23[...], 0.)
    xm2[...] = xm
    q2[...] = _dot(xm, bb2[...])
    p2[...] = _dot(xm, amb2[...]) + bg2[...]
    h2[...] = jnp.maximum(_dot(xm, g21[...]) + bg21[...], 0.)


def _tc3(s2, p2, bb3, amb3, bg3, sap1, q3, p3):
    sp = jnp.maximum(p2[...] + s2[...], 0.)
    sap1[...] = sp
    q3[...] = _dot(sp, bb3[...])
    p3[...] = _dot(sp, amb3[...]) + bg3[...]


def _tc4(s3, p3, h2, sap1, wa, wb, wc, bglm2, xg2):
    sap2 = jnp.maximum(p3[...] + s3[...], 0.)
    xg2[...] = jnp.maximum(
        _dot(h2[...], wa[...]) + _dot(sap1[...], wb[...])
        + _dot(sap2, wc[...]) + bglm2[...], 0.)


def _tc5(xm2, xg2, m31a, m31b, b31, w32, b32, w33, b33, w34, b34, h4):
    h = jnp.maximum(_dot(xm2[...], m31a[...]) + _dot(xg2[...], m31b[...])
                    + b31[...], 0.)
    h = jnp.maximum(_dot(h, w32[...]) + b32[...], 0.)
    h = jnp.maximum(_dot(h, w33[...]) + b33[...], 0.)
    h4[...] = jnp.maximum(_dot(h, w34[...]) + b34[...], 0.)


def _attn_pool(x, seg, gw, gb, nseg):
    gate = _dot(x, gw) + gb
    ids = lax.broadcasted_iota(jnp.int32, (1, nseg), 1)
    mask = (seg == ids).astype(jnp.float32)
    gmax = jnp.max(jnp.where(mask > 0., gate, -jnp.inf), axis=0, keepdims=True)
    m = jnp.where(jnp.isfinite(gmax), gmax, 0.)
    g = jnp.exp(gate - jnp.sum(mask * m, axis=1, keepdims=True))
    denom = jnp.sum(mask * g, axis=0, keepdims=True)
    gn = g / (jnp.sum(mask * denom, axis=1, keepdims=True) + 1e-16)
    pooled = _dot((mask * gn).T, x)
    return mask, pooled


def _pool_jaw(xg, jb, gw, gb, out):
    x = xg[...]
    mask, pooled = _attn_pool(x, jb[...], gw[...], gb[...], 2)
    out[...] = x + _dot(mask, pooled)


def _pool_teeth(h4, tb, gw, gb, out, tf):
    x = h4[...]
    mask, pooled = _attn_pool(x, tb[...], gw[...], gb[...], 32)
    tf[...] = pooled
    out[...] = x + _dot(mask, pooled)


def _head(h5, tf, tb, hmw, hmb, hgw, hgb, tp1, tbb1, tp2, tbb2, tp3, tbb3,
          abrw, abrb, hm_o, efs_o, abr_o, t_o):
    t = _dot(tf[...], tp1[...]) + tbb1[...]
    t = _dot(t, tp2[...]) + tbb2[...]
    t_o[...] = jax.nn.sigmoid(_dot(t, tp3[...]) + tbb3[...])
    hm = jax.nn.sigmoid(_dot(h5[...], hmw[...]) + hmb[...])
    hm_o[...] = hm
    _, ef = _attn_pool(hm, tb[...], hgw[...], hgb[...], 32)
    efs_o[...] = jax.nn.sigmoid(ef)
    abr_o[...] = jax.nn.sigmoid(_dot(ef, abrw[...]) + abrb[...])


def _ungridded_call(body, out_shapes):
    return pl.pallas_call(
        body,
        out_shape=[jax.ShapeDtypeStruct(s, jnp.float32) for s in out_shapes],
    )


def _wt(params, name):
    return params[name + "_W"].T, params[name + "_b"][None, :]


def _conv_wt(params, name, cin):
    wt = params[name + "_W"].T
    a, b = wt[:cin], wt[cin:]
    return b, a - b, params[name + "_b"][None, :]


def kernel(x, edge_index, jaw_batch, teeth_batch, params):
    src = edge_index[0]
    dst = edge_index[1]
    jb = jaw_batch[:, None]
    tb = teeth_batch[:, None]

    w11, b11 = _wt(params, "mlp1_1")
    w12, b12 = _wt(params, "mlp1_2")
    bb1, amb1, bg1 = _conv_wt(params, "gcn1", 64)
    bb1 = jnp.pad(bb1, ((0, 0), (0, 96)))
    g11, bg11 = _wt(params, "glm1_1")
    g12, bglm = _wt(params, "glm1_2")
    ga, gb_ = g12[:32], g12[32:]
    w21, b21 = _wt(params, "mlp2_1")
    w22, b22 = _wt(params, "mlp2_2")
    w23, b23 = _wt(params, "mlp2_3")
    bb2, amb2, bg2 = _conv_wt(params, "gcn2", 256)
    g21, bg21 = _wt(params, "glm2_1")
    bb3, amb3, bg3 = _conv_wt(params, "gcn3", 128)
    g22, bglm2 = _wt(params, "glm2_2")
    wa, wb, wc = g22[:128], g22[128:256], g22[256:]
    jgw, jgb = _wt(params, "jaw_gate")
    m31, b31 = _wt(params, "mlp3_1")
    m31a, m31b = m31[:256], m31[256:]
    w32, b32 = _wt(params, "mlp3_2")
    w33, b33 = _wt(params, "mlp3_3")
    w34, b34 = _wt(params, "mlp3_4")
    tgw, tgb = _wt(params, "teeth_gate")
    tp1, tbb1 = _wt(params, "tp1")
    tp2, tbb2 = _wt(params, "tp2")
    tp3, tbb3 = _wt(params, "tp3")
    hmw, hmb = _wt(params, "hm")
    hgw, hgb = _wt(params, "hm_gate")
    abrw, abrb = _wt(params, "abr")

    bsrc, bdl, cnt = _make_bucketize()(src, dst)
    _seg32 = _make_segmax(32)
    _seg128 = _make_segmax(128)

    q1, p1, h1 = _stage_call(
        _tc1, [x, w11, b11, w12, b12, bb1, amb1, bg1, g11, bg11],
        [15], [128, 32, 32])
    s1 = _seg32(q1, bsrc, bdl, cnt).reshape(NPAD, 32)[:N]

    xm2, q2, p2, h2 = _stage_call(
        _tc2, [s1, p1, h1, ga, gb_, bglm, w21, b21, w22, b22, w23, b23,
               bb2, amb2, bg2, g21, bg21],
        [32, 32, 32], [256, 128, 128, 128])
    s2 = _seg128(q2, bsrc, bdl, cnt).reshape(NPAD, 128)[:N]

    sap1, q3, p3 = _stage_call(
        _tc3, [s2, p2, bb3, amb3, bg3], [128, 128], [128, 128, 128])
    s3 = _seg128(q3, bsrc, bdl, cnt).reshape(NPAD, 128)[:N]

    xg2_pre, = _stage_call(
        _tc4, [s3, p3, h2, sap1, wa, wb, wc, bglm2],
        [128, 128, 128, 128], [256])

    xg2, = pl.pallas_call(
        _pool_jaw,
        out_shape=[jax.ShapeDtypeStruct((N, 256), jnp.float32)],
    )(xg2_pre, jb, jgw, jgb)

    h4, = _stage_call(
        _tc5, [xm2, xg2, m31a, m31b, b31, w32, b32, w33, b33, w34, b34],
        [256, 256], [128])

    h5, tf = pl.pallas_call(
        _pool_teeth,
        out_shape=[jax.ShapeDtypeStruct((N, 128), jnp.float32),
                   jax.ShapeDtypeStruct((32, 128), jnp.float32)],
    )(h4, tb, tgw, tgb)

    hm, efs, abr, t = pl.pallas_call(
        _head,
        out_shape=[jax.ShapeDtypeStruct((N, 5), jnp.float32),
                   jax.ShapeDtypeStruct((32, 5), jnp.float32),
                   jax.ShapeDtypeStruct((32, 1), jnp.float32),
                   jax.ShapeDtypeStruct((32, 3), jnp.float32)],
    )(h5, tf, tb, hmw, hmb, hgw, hgb, tp1, tbb1, tp2, tbb2, tp3, tbb3,
      abrw, abrb)

    landmark_heatmap = hm.T[None]
    landmark_exist_prob = efs.T[None]
    teeth_abrasion_prob = abr.T[None]
    teeth_twisted = t[:, 0].reshape(1, 1, -1)
    teeth_tilted = t[:, 1].reshape(1, 1, -1)
    teeth_ectopic = t[:, 2].reshape(1, 1, -1)
    return (landmark_heatmap, landmark_exist_prob, teeth_abrasion_prob,
            teeth_twisted, teeth_tilted, teeth_ectopic)


# speedup vs baseline: 6.0138x; 1.3349x over previous
import functools

import jax
import jax.numpy as jnp
from jax import lax
from jax.experimental import pallas as pl
from jax.experimental.pallas import tpu as pltpu
from jax.experimental.pallas import tpu_sc as plsc

N = 10000
E = 320000
NW = 32
RANGE = 313
NPAD = NW * RANGE
CAP = 321536
CHA = 6400
RING = 4096
HALF = 2048
CHB = 128
SUP = 2048

_MESH = dict(core_axis_name="c", subcore_axis_name="s")


def _wid():
    return lax.axis_index("s") * 2 + lax.axis_index("c")


@functools.cache
def _make_bucketize():
    return functools.partial(
        pl.kernel,
        out_type=[
            jax.ShapeDtypeStruct((NW, CAP), jnp.int32),
            jax.ShapeDtypeStruct((NW, CAP), jnp.int32),
            jax.ShapeDtypeStruct((NW, 16), jnp.int32),
        ],
        compiler_params=pltpu.CompilerParams(needs_layout_passes=False),
        mesh=plsc.VectorSubcoreMesh(**_MESH),
        scratch_types=[
            pltpu.VMEM((CHA,), jnp.int32),
            pltpu.VMEM((CHA,), jnp.int32),
            pltpu.VMEM((CHA,), jnp.int32),
            pltpu.VMEM((CHA,), jnp.int32),
            pltpu.VMEM((RING,), jnp.int32),
            pltpu.VMEM((RING,), jnp.int32),
            pltpu.VMEM((16,), jnp.int32),
            pltpu.SemaphoreType.DMA,
            pltpu.SemaphoreType.DMA,
        ],
    )(_bucketize_body)


def _bucketize_body(src_hbm, dst_hbm, bsrc, bdl, cnt,
                    src0, dst0, src1, dst1, ring_s, ring_d, cnt_v, sem0, sem1):
    w = _wid()
    lo = w * RANGE
    nch = E // CHA

    def ini(i, _):
        off = pl.multiple_of(i * 16, 16)
        ring_s[pl.ds(off, 16)] = jnp.zeros((16,), jnp.int32)
        ring_d[pl.ds(off, 16)] = jnp.full((16,), RANGE, jnp.int32)
        return 0
    lax.fori_loop(0, RING // 16, ini, 0)

    def flush(do, f):
        half0 = (f // HALF) % 2 == 0
        foff = pl.multiple_of(f, HALF)

        @pl.when(do & half0)
        def _():
            pltpu.sync_copy(ring_s.at[pl.ds(0, HALF)], bsrc.at[w, pl.ds(foff, HALF)])
            pltpu.sync_copy(ring_d.at[pl.ds(0, HALF)], bdl.at[w, pl.ds(foff, HALF)])

        @pl.when(do & jnp.logical_not(half0))
        def _():
            pltpu.sync_copy(ring_s.at[pl.ds(HALF, HALF)], bsrc.at[w, pl.ds(foff, HALF)])
            pltpu.sync_copy(ring_d.at[pl.ds(HALF, HALF)], bdl.at[w, pl.ds(foff, HALF)])
        return jnp.where(do, f + HALF, f)

    def load(g, src_b, dst_b, sem_b):
        @pl.when(g < nch)
        def _():
            goff = pl.multiple_of(g * CHA, 8)
            pltpu.async_copy(src_hbm.at[pl.ds(goff, CHA)], src_b, sem_b)
            pltpu.async_copy(dst_hbm.at[pl.ds(goff, CHA)], dst_b, sem_b)

    def scan(carry, src_b, dst_b, sem_b):
        cur_v, f = carry
        pltpu.make_async_copy(src_hbm.at[pl.ds(0, CHA)], src_b, sem_b).wait()
        pltpu.make_async_copy(dst_hbm.at[pl.ds(0, CHA)], dst_b, sem_b).wait()

        def inner(i, cv):
            for u in range(4):
                off = pl.multiple_of(i * 64 + u * 16, 16)
                sv = src_b[pl.ds(off, 16)]
                dv = dst_b[pl.ds(off, 16)]
                dvl = dv - lo
                m = dvl.astype(jnp.uint32) < jnp.uint32(RANGE)
                mi = m.astype(jnp.int32)
                ps = plsc.cumsum(mi)
                pos = (cv + ps - mi) & (RING - 1)
                plsc.store_scatter(ring_s, [pos], sv, mask=m)
                plsc.store_scatter(ring_d, [pos], dvl, mask=m)
                cv = cv + plsc.all_reduce_population_count(m)
            return cv

        cur_v = lax.fori_loop(0, CHA // 64, inner, cur_v)
        cur_s = jnp.max(cur_v)
        f = flush(cur_s - f >= HALF, f)
        return cur_v, f

    load(jnp.int32(0), src0, dst0, sem0)

    def pair(g2, carry):
        g = g2 * 2
        load(g + 1, src1, dst1, sem1)
        carry = scan(carry, src0, dst0, sem0)
        load(g + 2, src0, dst0, sem0)
        carry = scan(carry, src1, dst1, sem1)
        return carry

    cur_v, f = lax.fori_loop(
        0, nch // 2, pair, (jnp.zeros((16,), jnp.int32), jnp.int32(0)))
    flush(jnp.max(cur_v) > f, f)
    cnt_v[...] = cur_v
    pltpu.sync_copy(cnt_v, cnt.at[w])


@functools.cache
def _make_segmax(C):
    accw = (RANGE + 1) * C
    crow = 128

    @functools.partial(
        pl.kernel,
        out_type=jax.ShapeDtypeStruct((NPAD * C,), jnp.float32),
        compiler_params=pltpu.CompilerParams(needs_layout_passes=False),
        mesh=plsc.VectorSubcoreMesh(**_MESH),
        scratch_types=[
            pltpu.VMEM((SUP,), jnp.int32),
            pltpu.VMEM((SUP,), jnp.int32),
            pltpu.VMEM((SUP,), jnp.int32),
            pltpu.VMEM((SUP,), jnp.int32),
            pltpu.VMEM((CHB, crow), jnp.float32),
            pltpu.VMEM((CHB, crow), jnp.float32),
            pltpu.VMEM((accw,), jnp.float32),
            pltpu.VMEM((16,), jnp.int32),
            pltpu.SemaphoreType.DMA,
            pltpu.SemaphoreType.DMA,
            pltpu.SemaphoreType.DMA,
            pltpu.SemaphoreType.DMA,
        ],
    )
    def seg(q_hbm, bsrc, bdl, cnt, out_hbm,
            sidx0, sdl0, sidx1, sdl1, rows_a, rows_b, acc, nv,
            sem_l0, sem_l1, sem_a, sem_b):
        w = _wid()
        neg = jnp.full((16,), -jnp.inf, jnp.float32)

        def ini(i, _):
            acc[pl.ds(pl.multiple_of(i * 16, 16), 16)] = neg
            return 0
        lax.fori_loop(0, accw // 16, ini, 0)

        pltpu.sync_copy(cnt.at[w], nv)
        n = jnp.max(nv[...])
        nsup = (n + SUP - 1) // SUP

        def loadsup(s, sidx, sdl, sem_l):
            @pl.when(s < nsup)
            def _():
                soff = pl.multiple_of(s * SUP, 8)
                pltpu.async_copy(bsrc.at[w, pl.ds(soff, SUP)], sidx, sem_l)
                pltpu.async_copy(bdl.at[w, pl.ds(soff, SUP)], sdl, sem_l)

        def waitsup(sidx, sdl, sem_l):
            pltpu.make_async_copy(bsrc.at[w, pl.ds(0, SUP)], sidx, sem_l).wait()
            pltpu.make_async_copy(bdl.at[w, pl.ds(0, SUP)], sdl, sem_l).wait()

        def fire(sidx, p, rows, sem_g):
            poff = pl.multiple_of(p * CHB, CHB)
            pltpu.async_copy(q_hbm.at[sidx.at[pl.ds(poff, CHB)]], rows, sem_g)

        def proc(sidx, sdl, p, rows, sem_g):
            pltpu.make_async_copy(
                q_hbm.at[sidx.at[pl.ds(0, CHB)]], rows, sem_g).wait()

            def grp(j, _):
                doff = pl.multiple_of(p * CHB + j * 16, 16)
                dl16 = sdl[pl.ds(doff, 16)]
                for jj in range(16):
                    base = pl.multiple_of(dl16[jj] * C, C)
                    e = j * 16 + jj
                    for k in range(C // 16):
                        aoff = pl.multiple_of(base + k * 16, 16)
                        r = rows[e, pl.ds(k * 16, 16)]
                        acc[pl.ds(aoff, 16)] = jnp.maximum(acc[pl.ds(aoff, 16)], r)
                return 0
            lax.fori_loop(0, CHB // 16, grp, 0)

        def supstep(s, sidx, sdl, sem_l, o_sidx, o_sdl, o_sem_l):
            loadsup(s + 1, o_sidx, o_sdl, o_sem_l)
            waitsup(sidx, sdl, sem_l)
            nblk = jnp.minimum(SUP // CHB, (n - s * SUP + CHB - 1) // CHB)
            fire(sidx, jnp.int32(0), rows_a, sem_a)

            def blkpair(p2, _):
                p = p2 * 2

                @pl.when(p < nblk)
                def _():
                    @pl.when(p + 1 < nblk)
                    def _():
                        fire(sidx, p + 1, rows_b, sem_b)
                    proc(sidx, sdl, p, rows_a, sem_a)

                @pl.when(p + 1 < nblk)
                def _():
                    @pl.when(p + 2 < nblk)
                    def _():
                        fire(sidx, p + 2, rows_a, sem_a)
                    proc(sidx, sdl, p + 1, rows_b, sem_b)
                return 0

            lax.fori_loop(0, SUP // CHB // 2, blkpair, 0)

        loadsup(jnp.int32(0), sidx0, sdl0, sem_l0)

        def suppair(s2, _):
            s = s2 * 2

            @pl.when(s < nsup)
            def _():
                supstep(s, sidx0, sdl0, sem_l0, sidx1, sdl1, sem_l1)

            @pl.when(s + 1 < nsup)
            def _():
                supstep(s + 1, sidx1, sdl1, sem_l1, sidx0, sdl0, sem_l0)
            return 0

        lax.fori_loop(0, (nsup + 1) // 2, suppair, 0)
        ooff = pl.multiple_of(w * RANGE * C, 8)
        pltpu.sync_copy(acc.at[pl.ds(0, RANGE * C)], out_hbm.at[pl.ds(ooff, RANGE * C)])

    return seg


BR = 1000
GRID = N // BR


def _dot(a, b):
    return jnp.dot(a, b, preferred_element_type=jnp.float32)


def _rows_spec(c):
    return pl.BlockSpec((BR, c), lambda i: (i, 0))


def _full_spec(shape):
    return pl.BlockSpec(shape, lambda i: tuple(0 for _ in shape))


def _stage_call(body, in_arrays, row_in_dims, out_dims):
    n_rows = len(row_in_dims)
    in_specs = [_rows_spec(d) for d in row_in_dims]
    in_specs += [_full_spec(a.shape) for a in in_arrays[n_rows:]]
    return pl.pallas_call(
        body,
        grid=(GRID,),
        in_specs=in_specs,
        out_specs=[_rows_spec(d) for d in out_dims],
        out_shape=[jax.ShapeDtypeStruct((N, d), jnp.float32) for d in out_dims],
    )(*in_arrays)


def _tc1(x, w11, b11, w12, b12, bb1, amb1, bg1, g11, bg11, q1, p1, h1):
    h = jnp.maximum(_dot(x[...], w11[...]) + b11[...], 0.)
    h = jnp.maximum(_dot(h, w12[...]) + b12[...], 0.)
    q1[...] = _dot(h, bb1[...])
    p1[...] = _dot(h, amb1[...]) + bg1[...]
    h1[...] = jnp.maximum(_dot(h, g11[...]) + bg11[...], 0.)


def _tc2(s1, p1, h1, ga, gb, bglm, w21, b21, w22, b22, w23, b23,
         bb2, amb2, bg2, g21, bg21, xm2, q2, p2, h2):
    sap = jnp.maximum(p1[...] + s1[...], 0.)
    h = jnp.maximum(_dot(h1[...], ga[...]) + _dot(sap, gb[...]) + bglm[...], 0.)
    h = jnp.maximum(_dot(h, w21[...]) + b21[...], 0.)
    h = jnp.maximum(_dot(h, w22[...]) + b22[...], 0.)
    xm = jnp.maximum(_dot(h, w23[...]) + b23[...], 0.)
    xm2[...] = xm
    q2[...] = _dot(xm, bb2[...])
    p2[...] = _dot(xm, amb2[...]) + bg2[...]
    h2[...] = jnp.maximum(_dot(xm, g21[...]) + bg21[...], 0.)


def _tc3(s2, p2, bb3, amb3, bg3, sap1, q3, p3):
    sp = jnp.maximum(p2[...] + s2[...], 0.)
    sap1[...] = sp
    q3[...] = _dot(sp, bb3[...])
    p3[...] = _dot(sp, amb3[...]) + bg3[...]


def _tc4(s3, p3, h2, sap1, wa, wb, wc, bglm2, xg2):
    sap2 = jnp.maximum(p3[...] + s3[...], 0.)
    xg2[...] = jnp.maximum(
        _dot(h2[...], wa[...]) + _dot(sap1[...], wb[...])
        + _dot(sap2, wc[...]) + bglm2[...], 0.)


def _tc5(xm2, xg2, m31a, m31b, b31, w32, b32, w33, b33, w34, b34, h4):
    h = jnp.maximum(_dot(xm2[...], m31a[...]) + _dot(xg2[...], m31b[...])
                    + b31[...], 0.)
    h = jnp.maximum(_dot(h, w32[...]) + b32[...], 0.)
    h = jnp.maximum(_dot(h, w33[...]) + b33[...], 0.)
    h4[...] = jnp.maximum(_dot(h, w34[...]) + b34[...], 0.)


def _attn_pool(x, seg, gw, gb, nseg):
    gate = _dot(x, gw) + gb
    ids = lax.broadcasted_iota(jnp.int32, (1, nseg), 1)
    mask = (seg == ids).astype(jnp.float32)
    gmax = jnp.max(jnp.where(mask > 0., gate, -jnp.inf), axis=0, keepdims=True)
    m = jnp.where(jnp.isfinite(gmax), gmax, 0.)
    g = jnp.exp(gate - jnp.sum(mask * m, axis=1, keepdims=True))
    denom = jnp.sum(mask * g, axis=0, keepdims=True)
    gn = g / (jnp.sum(mask * denom, axis=1, keepdims=True) + 1e-16)
    pooled = _dot((mask * gn).T, x)
    return mask, pooled


def _pool_jaw(xg, jb, gw, gb, out):
    x = xg[...]
    mask, pooled = _attn_pool(x, jb[...], gw[...], gb[...], 2)
    out[...] = x + _dot(mask, pooled)


def _pool_teeth(h4, tb, gw, gb, out, tf):
    x = h4[...]
    mask, pooled = _attn_pool(x, tb[...], gw[...], gb[...], 32)
    tf[...] = pooled
    out[...] = x + _dot(mask, pooled)


def _head(h5, tf, tb, hmw, hmb, hgw, hgb, tp1, tbb1, tp2, tbb2, tp3, tbb3,
          abrw, abrb, hm_o, efs_o, abr_o, t_o):
    t = _dot(tf[...], tp1[...]) + tbb1[...]
    t = _dot(t, tp2[...]) + tbb2[...]
    t_o[...] = jax.nn.sigmoid(_dot(t, tp3[...]) + tbb3[...])
    hm = jax.nn.sigmoid(_dot(h5[...], hmw[...]) + hmb[...])
    hm_o[...] = hm
    _, ef = _attn_pool(hm, tb[...], hgw[...], hgb[...], 32)
    efs_o[...] = jax.nn.sigmoid(ef)
    abr_o[...] = jax.nn.sigmoid(_dot(ef, abrw[...]) + abrb[...])


def _ungridded_call(body, out_shapes):
    return pl.pallas_call(
        body,
        out_shape=[jax.ShapeDtypeStruct(s, jnp.float32) for s in out_shapes],
    )


def _wt(params, name):
    return params[name + "_W"].T, params[name + "_b"][None, :]


def _conv_wt(params, name, cin):
    wt = params[name + "_W"].T
    a, b = wt[:cin], wt[cin:]
    return b, a - b, params[name + "_b"][None, :]


def kernel(x, edge_index, jaw_batch, teeth_batch, params):
    src = edge_index[0]
    dst = edge_index[1]
    jb = jaw_batch[:, None]
    tb = teeth_batch[:, None]

    w11, b11 = _wt(params, "mlp1_1")
    w12, b12 = _wt(params, "mlp1_2")
    bb1, amb1, bg1 = _conv_wt(params, "gcn1", 64)
    bb1 = jnp.pad(bb1, ((0, 0), (0, 96)))
    g11, bg11 = _wt(params, "glm1_1")
    g12, bglm = _wt(params, "glm1_2")
    ga, gb_ = g12[:32], g12[32:]
    w21, b21 = _wt(params, "mlp2_1")
    w22, b22 = _wt(params, "mlp2_2")
    w23, b23 = _wt(params, "mlp2_3")
    bb2, amb2, bg2 = _conv_wt(params, "gcn2", 256)
    g21, bg21 = _wt(params, "glm2_1")
    bb3, amb3, bg3 = _conv_wt(params, "gcn3", 128)
    g22, bglm2 = _wt(params, "glm2_2")
    wa, wb, wc = g22[:128], g22[128:256], g22[256:]
    jgw, jgb = _wt(params, "jaw_gate")
    m31, b31 = _wt(params, "mlp3_1")
    m31a, m31b = m31[:256], m31[256:]
    w32, b32 = _wt(params, "mlp3_2")
    w33, b33 = _wt(params, "mlp3_3")
    w34, b34 = _wt(params, "mlp3_4")
    tgw, tgb = _wt(params, "teeth_gate")
    tp1, tbb1 = _wt(params, "tp1")
    tp2, tbb2 = _wt(params, "tp2")
    tp3, tbb3 = _wt(params, "tp3")
    hmw, hmb = _wt(params, "hm")
    hgw, hgb = _wt(params, "hm_gate")
    abrw, abrb = _wt(params, "abr")

    bsrc, bdl, cnt = _make_bucketize()(src, dst)
    _seg32 = _make_segmax(32)
    _seg128 = _make_segmax(128)

    q1, p1, h1 = _stage_call(
        _tc1, [x, w11, b11, w12, b12, bb1, amb1, bg1, g11, bg11],
        [15], [128, 32, 32])
    s1 = _seg32(q1, bsrc, bdl, cnt).reshape(NPAD, 32)[:N]

    xm2, q2, p2, h2 = _stage_call(
        _tc2, [s1, p1, h1, ga, gb_, bglm, w21, b21, w22, b22, w23, b23,
               bb2, amb2, bg2, g21, bg21],
        [32, 32, 32], [256, 128, 128, 128])
    s2 = _seg128(q2, bsrc, bdl, cnt).reshape(NPAD, 128)[:N]

    sap1, q3, p3 = _stage_call(
        _tc3, [s2, p2, bb3, amb3, bg3], [128, 128], [128, 128, 128])
    s3 = _seg128(q3, bsrc, bdl, cnt).reshape(NPAD, 128)[:N]

    xg2_pre, = _stage_call(
        _tc4, [s3, p3, h2, sap1, wa, wb, wc, bglm2],
        [128, 128, 128, 128], [256])

    xg2, = pl.pallas_call(
        _pool_jaw,
        out_shape=[jax.ShapeDtypeStruct((N, 256), jnp.float32)],
    )(xg2_pre, jb, jgw, jgb)

    h4, = _stage_call(
        _tc5, [xm2, xg2, m31a, m31b, b31, w32, b32, w33, b33, w34, b34],
        [256, 256], [128])

    h5, tf = pl.pallas_call(
        _pool_teeth,
        out_shape=[jax.ShapeDtypeStruct((N, 128), jnp.float32),
                   jax.ShapeDtypeStruct((32, 128), jnp.float32)],
    )(h4, tb, tgw, tgb)

    hm, efs, abr, t = pl.pallas_call(
        _head,
        out_shape=[jax.ShapeDtypeStruct((N, 5), jnp.float32),
                   jax.ShapeDtypeStruct((32, 5), jnp.float32),
                   jax.ShapeDtypeStruct((32, 1), jnp.float32),
                   jax.ShapeDtypeStruct((32, 3), jnp.float32)],
    )(h5, tf, tb, hmw, hmb, hgw, hgb, tp1, tbb1, tp2, tbb2, tp3, tbb3,
      abrw, abrb)

    landmark_heatmap = hm.T[None]
    landmark_exist_prob = efs.T[None]
    teeth_abrasion_prob = abr.T[None]
    teeth_twisted = t[:, 0].reshape(1, 1, -1)
    teeth_tilted = t[:, 1].reshape(1, 1, -1)
    teeth_ectopic = t[:, 2].reshape(1, 1, -1)
    return (landmark_heatmap, landmark_exist_prob, teeth_abrasion_prob,
            teeth_twisted, teeth_tilted, teeth_ectopic)


# speedup vs baseline: 8.9120x; 1.4819x over previous
import functools

import jax
import jax.numpy as jnp
from jax import lax
from jax.experimental import pallas as pl
from jax.experimental.pallas import tpu as pltpu
from jax.experimental.pallas import tpu_sc as plsc

N = 10000
E = 320000
NW = 32
RANGE = 313
NPAD = NW * RANGE
CAP = 321536
CHA = 6400
RING = 4096
HALF = 2048
CHB = 128
SUP = 2048

_MESH = dict(core_axis_name="c", subcore_axis_name="s")


def _wid():
    return lax.axis_index("s") * 2 + lax.axis_index("c")


@functools.cache
def _make_bucketize():
    return functools.partial(
        pl.kernel,
        out_type=[
            jax.ShapeDtypeStruct((NW, CAP), jnp.int32),
            jax.ShapeDtypeStruct((NW, CAP), jnp.int32),
            jax.ShapeDtypeStruct((NW, 16), jnp.int32),
        ],
        compiler_params=pltpu.CompilerParams(needs_layout_passes=False),
        mesh=plsc.VectorSubcoreMesh(**_MESH),
        scratch_types=[
            pltpu.VMEM((CHA,), jnp.int32),
            pltpu.VMEM((CHA,), jnp.int32),
            pltpu.VMEM((CHA,), jnp.int32),
            pltpu.VMEM((CHA,), jnp.int32),
            pltpu.VMEM((RING,), jnp.int32),
            pltpu.VMEM((RING,), jnp.int32),
            pltpu.VMEM((16,), jnp.int32),
            pltpu.SemaphoreType.DMA,
            pltpu.SemaphoreType.DMA,
        ],
    )(_bucketize_body)


def _bucketize_body(src_hbm, dst_hbm, bsrc, bdl, cnt,
                    src0, dst0, src1, dst1, ring_s, ring_d, cnt_v, sem0, sem1):
    w = _wid()
    lo = w * RANGE
    nch = E // CHA

    def ini(i, _):
        off = pl.multiple_of(i * 16, 16)
        ring_s[pl.ds(off, 16)] = jnp.zeros((16,), jnp.int32)
        ring_d[pl.ds(off, 16)] = jnp.full((16,), RANGE, jnp.int32)
        return 0
    lax.fori_loop(0, RING // 16, ini, 0)

    def flush(do, f):
        half0 = (f // HALF) % 2 == 0
        foff = pl.multiple_of(f, HALF)

        @pl.when(do & half0)
        def _():
            pltpu.sync_copy(ring_s.at[pl.ds(0, HALF)], bsrc.at[w, pl.ds(foff, HALF)])
            pltpu.sync_copy(ring_d.at[pl.ds(0, HALF)], bdl.at[w, pl.ds(foff, HALF)])

        @pl.when(do & jnp.logical_not(half0))
        def _():
            pltpu.sync_copy(ring_s.at[pl.ds(HALF, HALF)], bsrc.at[w, pl.ds(foff, HALF)])
            pltpu.sync_copy(ring_d.at[pl.ds(HALF, HALF)], bdl.at[w, pl.ds(foff, HALF)])
        return jnp.where(do, f + HALF, f)

    def load(g, src_b, dst_b, sem_b):
        @pl.when(g < nch)
        def _():
            goff = pl.multiple_of(g * CHA, 8)
            pltpu.async_copy(src_hbm.at[pl.ds(goff, CHA)], src_b, sem_b)
            pltpu.async_copy(dst_hbm.at[pl.ds(goff, CHA)], dst_b, sem_b)

    def scan(carry, src_b, dst_b, sem_b):
        cur_v, f = carry
        pltpu.make_async_copy(src_hbm.at[pl.ds(0, CHA)], src_b, sem_b).wait()
        pltpu.make_async_copy(dst_hbm.at[pl.ds(0, CHA)], dst_b, sem_b).wait()

        def inner(i, cv):
            for u in range(4):
                off = pl.multiple_of(i * 64 + u * 16, 16)
                sv = src_b[pl.ds(off, 16)]
                dv = dst_b[pl.ds(off, 16)]
                dvl = dv - lo
                m = dvl.astype(jnp.uint32) < jnp.uint32(RANGE)
                mi = m.astype(jnp.int32)
                ps = plsc.cumsum(mi)
                pos = (cv + ps - mi) & (RING - 1)
                plsc.store_scatter(ring_s, [pos], sv, mask=m)
                plsc.store_scatter(ring_d, [pos], dvl, mask=m)
                cv = cv + plsc.all_reduce_population_count(m)
            return cv

        cur_v = lax.fori_loop(0, CHA // 64, inner, cur_v)
        cur_s = jnp.max(cur_v)
        f = flush(cur_s - f >= HALF, f)
        return cur_v, f

    load(jnp.int32(0), src0, dst0, sem0)

    def pair(g2, carry):
        g = g2 * 2
        load(g + 1, src1, dst1, sem1)
        carry = scan(carry, src0, dst0, sem0)
        load(g + 2, src0, dst0, sem0)
        carry = scan(carry, src1, dst1, sem1)
        return carry

    cur_v, f = lax.fori_loop(
        0, nch // 2, pair, (jnp.zeros((16,), jnp.int32), jnp.int32(0)))
    flush(jnp.max(cur_v) > f, f)
    cnt_v[...] = cur_v
    pltpu.sync_copy(cnt_v, cnt.at[w])


@functools.cache
def _make_segmax(C):
    accw = (RANGE + 1) * C
    crow = 128

    @functools.partial(
        pl.kernel,
        out_type=jax.ShapeDtypeStruct((NPAD * C,), jnp.float32),
        compiler_params=pltpu.CompilerParams(needs_layout_passes=False),
        mesh=plsc.VectorSubcoreMesh(**_MESH),
        scratch_types=[
            pltpu.VMEM((SUP,), jnp.int32),
            pltpu.VMEM((SUP,), jnp.int32),
            pltpu.VMEM((SUP,), jnp.int32),
            pltpu.VMEM((SUP,), jnp.int32),
            pltpu.VMEM((CHB, crow), jnp.float32),
            pltpu.VMEM((CHB, crow), jnp.float32),
            pltpu.VMEM((accw,), jnp.float32),
            pltpu.VMEM((16,), jnp.int32),
            pltpu.SemaphoreType.DMA,
            pltpu.SemaphoreType.DMA,
            pltpu.SemaphoreType.DMA,
            pltpu.SemaphoreType.DMA,
        ],
    )
    def seg(q_hbm, bsrc, bdl, cnt, out_hbm,
            sidx0, sdl0, sidx1, sdl1, rows_a, rows_b, acc, nv,
            sem_l0, sem_l1, sem_a, sem_b):
        w = _wid()
        neg = jnp.full((16,), -jnp.inf, jnp.float32)

        def ini(i, _):
            acc[pl.ds(pl.multiple_of(i * 16, 16), 16)] = neg
            return 0
        lax.fori_loop(0, accw // 16, ini, 0)

        pltpu.sync_copy(cnt.at[w], nv)
        n = jnp.max(nv[...])
        nsup = (n + SUP - 1) // SUP

        def loadsup(s, sidx, sdl, sem_l):
            @pl.when(s < nsup)
            def _():
                soff = pl.multiple_of(s * SUP, 8)
                pltpu.async_copy(bsrc.at[w, pl.ds(soff, SUP)], sidx, sem_l)
                pltpu.async_copy(bdl.at[w, pl.ds(soff, SUP)], sdl, sem_l)

        def waitsup(sidx, sdl, sem_l):
            pltpu.make_async_copy(bsrc.at[w, pl.ds(0, SUP)], sidx, sem_l).wait()
            pltpu.make_async_copy(bdl.at[w, pl.ds(0, SUP)], sdl, sem_l).wait()

        def fire(sidx, p, rows, sem_g):
            poff = pl.multiple_of(p * CHB, CHB)
            pltpu.async_copy(q_hbm.at[sidx.at[pl.ds(poff, CHB)]], rows, sem_g)

        def proc(sidx, sdl, p, rows, sem_g):
            pltpu.make_async_copy(
                q_hbm.at[sidx.at[pl.ds(0, CHB)]], rows, sem_g).wait()

            def grp(j, _):
                doff = pl.multiple_of(p * CHB + j * 16, 16)
                dl16 = sdl[pl.ds(doff, 16)]
                for jj in range(16):
                    base = pl.multiple_of(dl16[jj] * C, C)
                    e = j * 16 + jj
                    nk = C // 16
                    rs = [rows[e, pl.ds(k * 16, 16)] for k in range(nk)]
                    olds = [acc[pl.ds(pl.multiple_of(base + k * 16, 16), 16)]
                            for k in range(nk)]
                    for k in range(nk):
                        aoff = pl.multiple_of(base + k * 16, 16)
                        acc[pl.ds(aoff, 16)] = jnp.maximum(olds[k], rs[k])
                return 0
            lax.fori_loop(0, CHB // 16, grp, 0)

        def supstep(s, sidx, sdl, sem_l, o_sidx, o_sdl, o_sem_l):
            loadsup(s + 1, o_sidx, o_sdl, o_sem_l)
            waitsup(sidx, sdl, sem_l)
            nblk = jnp.minimum(SUP // CHB, (n - s * SUP + CHB - 1) // CHB)
            fire(sidx, jnp.int32(0), rows_a, sem_a)

            def blkpair(p2, _):
                p = p2 * 2

                @pl.when(p < nblk)
                def _():
                    @pl.when(p + 1 < nblk)
                    def _():
                        fire(sidx, p + 1, rows_b, sem_b)
                    proc(sidx, sdl, p, rows_a, sem_a)

                @pl.when(p + 1 < nblk)
                def _():
                    @pl.when(p + 2 < nblk)
                    def _():
                        fire(sidx, p + 2, rows_a, sem_a)
                    proc(sidx, sdl, p + 1, rows_b, sem_b)
                return 0

            lax.fori_loop(0, SUP // CHB // 2, blkpair, 0)

        loadsup(jnp.int32(0), sidx0, sdl0, sem_l0)

        def suppair(s2, _):
            s = s2 * 2

            @pl.when(s < nsup)
            def _():
                supstep(s, sidx0, sdl0, sem_l0, sidx1, sdl1, sem_l1)

            @pl.when(s + 1 < nsup)
            def _():
                supstep(s + 1, sidx1, sdl1, sem_l1, sidx0, sdl0, sem_l0)
            return 0

        lax.fori_loop(0, (nsup + 1) // 2, suppair, 0)
        ooff = pl.multiple_of(w * RANGE * C, 8)
        pltpu.sync_copy(acc.at[pl.ds(0, RANGE * C)], out_hbm.at[pl.ds(ooff, RANGE * C)])

    return seg


BR = 1000
GRID = N // BR


def _dot(a, b):
    return jnp.dot(a, b, preferred_element_type=jnp.float32)


def _rows_spec(c):
    return pl.BlockSpec((BR, c), lambda i: (i, 0))


def _full_spec(shape):
    return pl.BlockSpec(shape, lambda i: tuple(0 for _ in shape))


def _stage_call(body, in_arrays, row_in_dims, out_dims):
    n_rows = len(row_in_dims)
    in_specs = [_rows_spec(d) for d in row_in_dims]
    in_specs += [_full_spec(a.shape) for a in in_arrays[n_rows:]]
    return pl.pallas_call(
        body,
        grid=(GRID,),
        in_specs=in_specs,
        out_specs=[_rows_spec(d) for d in out_dims],
        out_shape=[jax.ShapeDtypeStruct((N, d), jnp.float32) for d in out_dims],
    )(*in_arrays)


def _tc1(x, w11, b11, w12, b12, bb1, amb1, bg1, g11, bg11, q1, p1, h1):
    h = jnp.maximum(_dot(x[...], w11[...]) + b11[...], 0.)
    h = jnp.maximum(_dot(h, w12[...]) + b12[...], 0.)
    q1[...] = _dot(h, bb1[...])
    p1[...] = _dot(h, amb1[...]) + bg1[...]
    h1[...] = jnp.maximum(_dot(h, g11[...]) + bg11[...], 0.)


def _tc2(s1, p1, h1, ga, gb, bglm, w21, b21, w22, b22, w23, b23,
         bb2, amb2, bg2, g21, bg21, xm2, q2, p2, h2):
    sap = jnp.maximum(p1[...] + s1[...], 0.)
    h = jnp.maximum(_dot(h1[...], ga[...]) + _dot(sap, gb[...]) + bglm[...], 0.)
    h = jnp.maximum(_dot(h, w21[...]) + b21[...], 0.)
    h = jnp.maximum(_dot(h, w22[...]) + b22[...], 0.)
    xm = jnp.maximum(_dot(h, w23[...]) + b23[...], 0.)
    xm2[...] = xm
    q2[...] = _dot(xm, bb2[...])
    p2[...] = _dot(xm, amb2[...]) + bg2[...]
    h2[...] = jnp.maximum(_dot(xm, g21[...]) + bg21[...], 0.)


def _tc3(s2, p2, bb3, amb3, bg3, sap1, q3, p3):
    sp = jnp.maximum(p2[...] + s2[...], 0.)
    sap1[...] = sp
    q3[...] = _dot(sp, bb3[...])
    p3[...] = _dot(sp, amb3[...]) + bg3[...]


def _tc4(s3, p3, h2, sap1, wa, wb, wc, bglm2, xg2):
    sap2 = jnp.maximum(p3[...] + s3[...], 0.)
    xg2[...] = jnp.maximum(
        _dot(h2[...], wa[...]) + _dot(sap1[...], wb[...])
        + _dot(sap2, wc[...]) + bglm2[...], 0.)


def _tc5(xm2, xg2, m31a, m31b, b31, w32, b32, w33, b33, w34, b34, h4):
    h = jnp.maximum(_dot(xm2[...], m31a[...]) + _dot(xg2[...], m31b[...])
                    + b31[...], 0.)
    h = jnp.maximum(_dot(h, w32[...]) + b32[...], 0.)
    h = jnp.maximum(_dot(h, w33[...]) + b33[...], 0.)
    h4[...] = jnp.maximum(_dot(h, w34[...]) + b34[...], 0.)


def _attn_pool(x, seg, gw, gb, nseg):
    gate = _dot(x, gw) + gb
    ids = lax.broadcasted_iota(jnp.int32, (1, nseg), 1)
    mask = (seg == ids).astype(jnp.float32)
    gmax = jnp.max(jnp.where(mask > 0., gate, -jnp.inf), axis=0, keepdims=True)
    m = jnp.where(jnp.isfinite(gmax), gmax, 0.)
    g = jnp.exp(gate - jnp.sum(mask * m, axis=1, keepdims=True))
    denom = jnp.sum(mask * g, axis=0, keepdims=True)
    gn = g / (jnp.sum(mask * denom, axis=1, keepdims=True) + 1e-16)
    pooled = _dot((mask * gn).T, x)
    return mask, pooled


def _pool_jaw(xg, jb, gw, gb, out):
    x = xg[...]
    mask, pooled = _attn_pool(x, jb[...], gw[...], gb[...], 2)
    out[...] = x + _dot(mask, pooled)


def _pool_teeth(h4, tb, gw, gb, out, tf):
    x = h4[...]
    mask, pooled = _attn_pool(x, tb[...], gw[...], gb[...], 32)
    tf[...] = pooled
    out[...] = x + _dot(mask, pooled)


def _head(h5, tf, tb, hmw, hmb, hgw, hgb, tp1, tbb1, tp2, tbb2, tp3, tbb3,
          abrw, abrb, hm_o, efs_o, abr_o, t_o):
    t = _dot(tf[...], tp1[...]) + tbb1[...]
    t = _dot(t, tp2[...]) + tbb2[...]
    t_o[...] = jax.nn.sigmoid(_dot(t, tp3[...]) + tbb3[...])
    hm = jax.nn.sigmoid(_dot(h5[...], hmw[...]) + hmb[...])
    hm_o[...] = hm
    _, ef = _attn_pool(hm, tb[...], hgw[...], hgb[...], 32)
    efs_o[...] = jax.nn.sigmoid(ef)
    abr_o[...] = jax.nn.sigmoid(_dot(ef, abrw[...]) + abrb[...])


def _ungridded_call(body, out_shapes):
    return pl.pallas_call(
        body,
        out_shape=[jax.ShapeDtypeStruct(s, jnp.float32) for s in out_shapes],
    )


def _wt(params, name):
    return params[name + "_W"].T, params[name + "_b"][None, :]


def _conv_wt(params, name, cin):
    wt = params[name + "_W"].T
    a, b = wt[:cin], wt[cin:]
    return b, a - b, params[name + "_b"][None, :]


def kernel(x, edge_index, jaw_batch, teeth_batch, params):
    src = edge_index[0]
    dst = edge_index[1]
    jb = jaw_batch[:, None]
    tb = teeth_batch[:, None]

    w11, b11 = _wt(params, "mlp1_1")
    w12, b12 = _wt(params, "mlp1_2")
    bb1, amb1, bg1 = _conv_wt(params, "gcn1", 64)
    bb1 = jnp.pad(bb1, ((0, 0), (0, 96)))
    g11, bg11 = _wt(params, "glm1_1")
    g12, bglm = _wt(params, "glm1_2")
    ga, gb_ = g12[:32], g12[32:]
    w21, b21 = _wt(params, "mlp2_1")
    w22, b22 = _wt(params, "mlp2_2")
    w23, b23 = _wt(params, "mlp2_3")
    bb2, amb2, bg2 = _conv_wt(params, "gcn2", 256)
    g21, bg21 = _wt(params, "glm2_1")
    bb3, amb3, bg3 = _conv_wt(params, "gcn3", 128)
    g22, bglm2 = _wt(params, "glm2_2")
    wa, wb, wc = g22[:128], g22[128:256], g22[256:]
    jgw, jgb = _wt(params, "jaw_gate")
    m31, b31 = _wt(params, "mlp3_1")
    m31a, m31b = m31[:256], m31[256:]
    w32, b32 = _wt(params, "mlp3_2")
    w33, b33 = _wt(params, "mlp3_3")
    w34, b34 = _wt(params, "mlp3_4")
    tgw, tgb = _wt(params, "teeth_gate")
    tp1, tbb1 = _wt(params, "tp1")
    tp2, tbb2 = _wt(params, "tp2")
    tp3, tbb3 = _wt(params, "tp3")
    hmw, hmb = _wt(params, "hm")
    hgw, hgb = _wt(params, "hm_gate")
    abrw, abrb = _wt(params, "abr")

    bsrc, bdl, cnt = _make_bucketize()(src, dst)
    _seg32 = _make_segmax(32)
    _seg128 = _make_segmax(128)

    q1, p1, h1 = _stage_call(
        _tc1, [x, w11, b11, w12, b12, bb1, amb1, bg1, g11, bg11],
        [15], [128, 32, 32])
    s1 = _seg32(q1, bsrc, bdl, cnt).reshape(NPAD, 32)[:N]

    xm2, q2, p2, h2 = _stage_call(
        _tc2, [s1, p1, h1, ga, gb_, bglm, w21, b21, w22, b22, w23, b23,
               bb2, amb2, bg2, g21, bg21],
        [32, 32, 32], [256, 128, 128, 128])
    s2 = _seg128(q2, bsrc, bdl, cnt).reshape(NPAD, 128)[:N]

    sap1, q3, p3 = _stage_call(
        _tc3, [s2, p2, bb3, amb3, bg3], [128, 128], [128, 128, 128])
    s3 = _seg128(q3, bsrc, bdl, cnt).reshape(NPAD, 128)[:N]

    xg2_pre, = _stage_call(
        _tc4, [s3, p3, h2, sap1, wa, wb, wc, bglm2],
        [128, 128, 128, 128], [256])

    xg2, = pl.pallas_call(
        _pool_jaw,
        out_shape=[jax.ShapeDtypeStruct((N, 256), jnp.float32)],
    )(xg2_pre, jb, jgw, jgb)

    h4, = _stage_call(
        _tc5, [xm2, xg2, m31a, m31b, b31, w32, b32, w33, b33, w34, b34],
        [256, 256], [128])

    h5, tf = pl.pallas_call(
        _pool_teeth,
        out_shape=[jax.ShapeDtypeStruct((N, 128), jnp.float32),
                   jax.ShapeDtypeStruct((32, 128), jnp.float32)],
    )(h4, tb, tgw, tgb)

    hm, efs, abr, t = pl.pallas_call(
        _head,
        out_shape=[jax.ShapeDtypeStruct((N, 5), jnp.float32),
                   jax.ShapeDtypeStruct((32, 5), jnp.float32),
                   jax.ShapeDtypeStruct((32, 1), jnp.float32),
                   jax.ShapeDtypeStruct((32, 3), jnp.float32)],
    )(h5, tf, tb, hmw, hmb, hgw, hgb, tp1, tbb1, tp2, tbb2, tp3, tbb3,
      abrw, abrb)

    landmark_heatmap = hm.T[None]
    landmark_exist_prob = efs.T[None]
    teeth_abrasion_prob = abr.T[None]
    teeth_twisted = t[:, 0].reshape(1, 1, -1)
    teeth_tilted = t[:, 1].reshape(1, 1, -1)
    teeth_ectopic = t[:, 2].reshape(1, 1, -1)
    return (landmark_heatmap, landmark_exist_prob, teeth_abrasion_prob,
            teeth_twisted, teeth_tilted, teeth_ectopic)
